# baseline (device time: 86981 ns/iter reference)
import math

import jax
import jax.numpy as jnp
from jax import lax
from jax.experimental import pallas as pl
from jax.experimental.pallas import tpu as pltpu

N_DEV = 16
B = 2
SQ = 128
S_TOT = N_DEV * SQ
D = 512
HQ = 4
DH = 64
HD = HQ * DH
SCALE = 0.125
LOG_BASE = math.log(10000.0)


def kernel(x, Wq, Wk, Wv, Wo):
    def body(x_ref, wq_ref, wk_ref, wv_ref, wo_ref, out_ref,
             kbuf, vbuf, ksend, krecv, vsend, vrecv):
        my_pos = lax.axis_index("i")
        left = (my_pos - 1) % N_DEV
        right = (my_pos + 1) % N_DEV

        barrier_sem = pltpu.get_barrier_semaphore()
        for nbr in (left, right):
            pl.semaphore_signal(
                barrier_sem, inc=1,
                device_id=(nbr,), device_id_type=pl.DeviceIdType.MESH,
            )
        pl.semaphore_wait(barrier_sem, 2)

        row = lax.broadcasted_iota(jnp.int32, (SQ, HD), 0)
        col = lax.broadcasted_iota(jnp.int32, (SQ, HD), 1)
        d_in_head = col % DH
        expnt = (2 * (d_in_head // 2)).astype(jnp.float32) / DH
        inv = jnp.exp(-expnt * LOG_BASE)
        pos = (my_pos * SQ + row).astype(jnp.float32)
        ang = pos * inv
        cosf = jnp.cos(ang)
        sinf = jnp.sin(ang)

        kk = lax.broadcasted_iota(jnp.int32, (HD, HD), 0)
        jj = lax.broadcasted_iota(jnp.int32, (HD, HD), 1)
        rot = jnp.where(
            (kk == jj + 1) & (jj % 2 == 0), -1.0,
            jnp.where((kk == jj - 1) & (jj % 2 == 1), 1.0, 0.0),
        ).astype(jnp.float32)

        wq = wq_ref[...].astype(jnp.bfloat16)
        wk = wk_ref[...].astype(jnp.bfloat16)
        wv = wv_ref[...].astype(jnp.bfloat16)
        wo = wo_ref[...].astype(jnp.bfloat16)

        qs = []
        for b in range(B):
            xb = x_ref[b].astype(jnp.bfloat16)
            q_raw = jnp.dot(xb, wq, preferred_element_type=jnp.float32)
            k_raw = jnp.dot(xb, wk, preferred_element_type=jnp.float32)
            v = jnp.dot(xb, wv, preferred_element_type=jnp.float32)
            q = q_raw * cosf + jnp.dot(
                q_raw, rot, preferred_element_type=jnp.float32) * sinf
            k = k_raw * cosf + jnp.dot(
                k_raw, rot, preferred_element_type=jnp.float32) * sinf
            qs.append(q)
            kbuf[pl.ds(my_pos * SQ, SQ), b * HD:(b + 1) * HD] = k.astype(jnp.bfloat16)
            vbuf[pl.ds(my_pos * SQ, SQ), b * HD:(b + 1) * HD] = v.astype(jnp.bfloat16)

        for h in range(N_DEV - 1):
            slot = (my_pos - h) % N_DEV
            k_rdma = pltpu.make_async_remote_copy(
                src_ref=kbuf.at[pl.ds(slot * SQ, SQ)],
                dst_ref=kbuf.at[pl.ds(slot * SQ, SQ)],
                send_sem=ksend.at[h],
                recv_sem=krecv.at[h],
                device_id=(right,),
                device_id_type=pl.DeviceIdType.MESH,
            )
            v_rdma = pltpu.make_async_remote_copy(
                src_ref=vbuf.at[pl.ds(slot * SQ, SQ)],
                dst_ref=vbuf.at[pl.ds(slot * SQ, SQ)],
                send_sem=vsend.at[h],
                recv_sem=vrecv.at[h],
                device_id=(right,),
                device_id_type=pl.DeviceIdType.MESH,
            )
            k_rdma.start()
            v_rdma.start()
            k_rdma.wait()
            v_rdma.wait()

        for b in range(B):
            ctxs = []
            for hh in range(HQ):
                c0 = b * HD + hh * DH
                q_h = qs[b][:, hh * DH:(hh + 1) * DH].astype(jnp.bfloat16)
                k_h = kbuf[:, c0:c0 + DH]
                s = lax.dot_general(
                    q_h, k_h, (((1,), (1,)), ((), ())),
                    preferred_element_type=jnp.float32) * SCALE
                m = jnp.max(s, axis=1, keepdims=True)
                e = jnp.exp(s - m)
                p = e / jnp.sum(e, axis=1, keepdims=True)
                v_h = vbuf[:, c0:c0 + DH]
                ctxs.append(jnp.dot(p.astype(jnp.bfloat16), v_h,
                                    preferred_element_type=jnp.float32))
            ctx_b = jnp.concatenate(ctxs, axis=1).astype(jnp.bfloat16)
            out_ref[b] = jnp.dot(ctx_b, wo, preferred_element_type=jnp.float32)

    return pl.pallas_call(
        body,
        out_shape=jax.ShapeDtypeStruct((B, SQ, D), jnp.float32),
        in_specs=[pl.BlockSpec(memory_space=pltpu.VMEM)] * 5,
        out_specs=pl.BlockSpec(memory_space=pltpu.VMEM),
        scratch_shapes=[
            pltpu.VMEM((S_TOT, B * HD), jnp.bfloat16),
            pltpu.VMEM((S_TOT, B * HD), jnp.bfloat16),
            pltpu.SemaphoreType.DMA((N_DEV - 1,)),
            pltpu.SemaphoreType.DMA((N_DEV - 1,)),
            pltpu.SemaphoreType.DMA((N_DEV - 1,)),
            pltpu.SemaphoreType.DMA((N_DEV - 1,)),
        ],
        compiler_params=pltpu.CompilerParams(collective_id=0),
    )(x, Wq, Wk, Wv, Wo)


# device time: 58150 ns/iter; 1.4958x vs baseline; 1.4958x over previous
import math

import jax
import jax.numpy as jnp
from jax import lax
from jax.experimental import pallas as pl
from jax.experimental.pallas import tpu as pltpu

N_DEV = 16
B = 2
SQ = 128
S_TOT = N_DEV * SQ
D = 512
HQ = 4
DH = 64
HD = HQ * DH
KV_W = 2 * B * HD
V0 = B * HD
SCALE = 0.125
LOG_BASE = math.log(10000.0)


def kernel(x, Wq, Wk, Wv, Wo):
    def body(x_ref, wq_ref, wk_ref, wv_ref, wo_ref, out_ref,
             kvbuf, send_sems, recv_sems):
        my_pos = lax.axis_index("i")

        barrier_sem = pltpu.get_barrier_semaphore()
        for d in range(1, N_DEV):
            pl.semaphore_signal(
                barrier_sem, inc=1,
                device_id=((my_pos + d) % N_DEV,),
                device_id_type=pl.DeviceIdType.MESH,
            )
        pl.semaphore_wait(barrier_sem, N_DEV - 1)

        row = lax.broadcasted_iota(jnp.int32, (SQ, HD), 0)
        col = lax.broadcasted_iota(jnp.int32, (SQ, HD), 1)
        d_in_head = col % DH
        expnt = (2 * (d_in_head // 2)).astype(jnp.float32) / DH
        inv = jnp.exp(-expnt * LOG_BASE)
        pos = (my_pos * SQ + row).astype(jnp.float32)
        ang = pos * inv
        cosf = jnp.cos(ang)
        sinf = jnp.sin(ang)

        kk = lax.broadcasted_iota(jnp.int32, (HD, HD), 0)
        jj = lax.broadcasted_iota(jnp.int32, (HD, HD), 1)
        rot = jnp.where(
            (kk == jj + 1) & (jj % 2 == 0), -1.0,
            jnp.where((kk == jj - 1) & (jj % 2 == 1), 1.0, 0.0),
        ).astype(jnp.float32)

        wq = wq_ref[...].astype(jnp.bfloat16)
        wk = wk_ref[...].astype(jnp.bfloat16)
        wv = wv_ref[...].astype(jnp.bfloat16)
        wo = wo_ref[...].astype(jnp.bfloat16)

        xbs = [x_ref[b].astype(jnp.bfloat16) for b in range(B)]
        for b in range(B):
            k_raw = jnp.dot(xbs[b], wk, preferred_element_type=jnp.float32)
            v = jnp.dot(xbs[b], wv, preferred_element_type=jnp.float32)
            k = k_raw * cosf + jnp.dot(
                k_raw, rot, preferred_element_type=jnp.float32) * sinf
            kvbuf[pl.ds(my_pos * SQ, SQ), b * HD:(b + 1) * HD] = k.astype(jnp.bfloat16)
            kvbuf[pl.ds(my_pos * SQ, SQ), V0 + b * HD:V0 + (b + 1) * HD] = (
                v.astype(jnp.bfloat16))

        rdmas = []
        for d in range(1, N_DEV):
            tgt = (my_pos + d) % N_DEV
            rdma = pltpu.make_async_remote_copy(
                src_ref=kvbuf.at[pl.ds(my_pos * SQ, SQ)],
                dst_ref=kvbuf.at[pl.ds(my_pos * SQ, SQ)],
                send_sem=send_sems.at[d - 1],
                recv_sem=recv_sems.at[d - 1],
                device_id=(tgt,),
                device_id_type=pl.DeviceIdType.MESH,
            )
            rdma.start()
            rdmas.append(rdma)

        qs = []
        for b in range(B):
            q_raw = jnp.dot(xbs[b], wq, preferred_element_type=jnp.float32)
            q = q_raw * cosf + jnp.dot(
                q_raw, rot, preferred_element_type=jnp.float32) * sinf
            qs.append(q)

        for rdma in rdmas:
            rdma.wait_recv()

        for b in range(B):
            ctxs = []
            for hh in range(HQ):
                c0 = b * HD + hh * DH
                q_h = qs[b][:, hh * DH:(hh + 1) * DH].astype(jnp.bfloat16)
                k_h = kvbuf[:, c0:c0 + DH]
                s = lax.dot_general(
                    q_h, k_h, (((1,), (1,)), ((), ())),
                    preferred_element_type=jnp.float32) * SCALE
                m = jnp.max(s, axis=1, keepdims=True)
                e = jnp.exp(s - m)
                p = e / jnp.sum(e, axis=1, keepdims=True)
                v_h = kvbuf[:, V0 + c0:V0 + c0 + DH]
                ctxs.append(jnp.dot(p.astype(jnp.bfloat16), v_h,
                                    preferred_element_type=jnp.float32))
            ctx_b = jnp.concatenate(ctxs, axis=1).astype(jnp.bfloat16)
            out_ref[b] = jnp.dot(ctx_b, wo, preferred_element_type=jnp.float32)

        for rdma in rdmas:
            rdma.wait_send()

    return pl.pallas_call(
        body,
        out_shape=jax.ShapeDtypeStruct((B, SQ, D), jnp.float32),
        in_specs=[pl.BlockSpec(memory_space=pltpu.VMEM)] * 5,
        out_specs=pl.BlockSpec(memory_space=pltpu.VMEM),
        scratch_shapes=[
            pltpu.VMEM((S_TOT, KV_W), jnp.bfloat16),
            pltpu.SemaphoreType.DMA((N_DEV - 1,)),
            pltpu.SemaphoreType.DMA((N_DEV - 1,)),
        ],
        compiler_params=pltpu.CompilerParams(collective_id=0),
    )(x, Wq, Wk, Wv, Wo)
